# Initial kernel scaffold; baseline (speedup 1.0000x reference)
import jax
import jax.numpy as jnp
from jax.experimental import pallas as pl


def kernel(x, edge_attr, node_W, node_b, edge_W, edge_b, t, conv_W1, conv_b1, conv_ln_g, conv_ln_b, conv_W2, conv_b2, layer_ln_g, layer_ln_b, lin_W, lin_b, edge_index):
    raise NotImplementedError("write your pallas kernel here")


# speedup vs baseline: 4.0893x; 4.0893x over previous
import functools

import jax
import jax.numpy as jnp
from jax import lax
from jax.experimental import pallas as pl
from jax.experimental.pallas import tpu as pltpu
from jax.experimental.pallas import tpu_sc as plsc

N = 10000
E = 320000
NUM_FEAT = 128
HID = 64
EXP = 128
L = 12
EPS = 1e-7

NC = 2
NS = 16
NW = NC * NS
EPW = E // NW
CH = 80
NCHUNK = EPW // CH
NP = 10240
RPW = NP // NS


def _sc_agg_body(ztab_ref, ea_ref, src_ref, dst_ref, t_ref, zero_ref,
                 out_ref, src_v, dst_v, h_buf, ea_buf, out_buf, tv, acc, sem):
    cid = lax.axis_index("c")
    sid = lax.axis_index("s")
    wid = cid * NS + sid

    pltpu.sync_copy(t_ref, tv)
    pltpu.sync_copy(zero_ref, acc.at[pl.ds(sid * RPW, RPW)])
    plsc.subcore_barrier()

    tval = tv[...]

    def chunk_body(i, _):
        base = wid * EPW + i * CH
        pltpu.sync_copy(src_ref.at[pl.ds(base, CH)], src_v)
        pltpu.sync_copy(dst_ref.at[pl.ds(base, CH)], dst_v)
        pltpu.async_copy(ztab_ref.at[src_v], h_buf, sem).wait()
        pltpu.sync_copy(ea_ref.at[pl.ds(base, CH)], ea_buf)

        def row_body(r, _):
            for g in range(4):
                c = g * 16
                hv = h_buf[r, pl.ds(c, 16)]
                ev = ea_buf[r, pl.ds(c, 16)]
                m = jnp.maximum(hv + ev, 0.0) + EPS
                ex = jnp.exp(tval * m)
                out_buf[r, pl.ds(c, 16)] = ex
                out_buf[r, pl.ds(64 + c, 16)] = ex * m
            return 0

        lax.fori_loop(0, CH, row_body, 0, unroll=2)
        pltpu.sync_copy(out_buf, acc.at[dst_v], add=True)
        return 0

    lax.fori_loop(0, NCHUNK, chunk_body, 0)
    plsc.subcore_barrier()
    pltpu.sync_copy(acc.at[pl.ds(sid * RPW, RPW)],
                    out_ref.at[pl.ds(cid * NP + sid * RPW, RPW)])


def _sc_aggregate(ztab, ea, src, dst, tvec, zero_block):
    kern = pl.kernel(
        _sc_agg_body,
        out_type=jax.ShapeDtypeStruct((NC * NP, 128), jnp.float32),
        mesh=plsc.VectorSubcoreMesh(core_axis_name="c", subcore_axis_name="s",
                                    num_cores=NC, num_subcores=NS),
        scratch_types=[
            pltpu.VMEM((CH,), jnp.int32),
            pltpu.VMEM((CH,), jnp.int32),
            pltpu.VMEM((CH, HID), jnp.float32),
            pltpu.VMEM((CH, HID), jnp.float32),
            pltpu.VMEM((CH, 128), jnp.float32),
            pltpu.VMEM((16,), jnp.float32),
            pltpu.VMEM_SHARED((NP, 128), jnp.float32),
            pltpu.SemaphoreType.DMA,
        ],
        compiler_params=pltpu.CompilerParams(use_tc_tiling_on_sc=False),
    )
    return kern(ztab, ea, src, dst, tvec, zero_block)


BN = 1000
NB = N // BN
EDGE_DIM = 4


def _ln(u, g, b):
    mu = jnp.mean(u, axis=-1, keepdims=True)
    var = jnp.mean((u - mu) ** 2, axis=-1, keepdims=True)
    return (u - mu) / jnp.sqrt(var + 1e-5) * g + b


def _node_encode_body(x_ref, w_ref, b_ref, o_ref):
    o_ref[...] = jnp.dot(x_ref[...], w_ref[...],
                         preferred_element_type=jnp.float32) + b_ref[...]


def _node_encode(x, w, b):
    return pl.pallas_call(
        _node_encode_body,
        out_shape=jax.ShapeDtypeStruct((N, HID), jnp.float32),
        grid=(NB,),
        in_specs=[
            pl.BlockSpec((BN, NUM_FEAT), lambda i: (i, 0)),
            pl.BlockSpec((NUM_FEAT, HID), lambda i: (0, 0)),
            pl.BlockSpec((1, HID), lambda i: (0, 0)),
        ],
        out_specs=pl.BlockSpec((BN, HID), lambda i: (i, 0)),
    )(x, w, b)


BE = 16000
NEB = E // BE


def _edge_encode_body(a_ref, w_ref, b_ref, o_ref):
    a = a_ref[...]
    w = w_ref[...]
    acc = jnp.broadcast_to(b_ref[...], (BE, HID))
    for k in range(EDGE_DIM):
        acc = acc + a[:, k:k + 1] * w[k:k + 1, :]
    o_ref[...] = acc


def _edge_encode(ea, w, b):
    return pl.pallas_call(
        _edge_encode_body,
        out_shape=jax.ShapeDtypeStruct((E, HID), jnp.float32),
        grid=(NEB,),
        in_specs=[
            pl.BlockSpec((BE, EDGE_DIM), lambda i: (i, 0)),
            pl.BlockSpec((EDGE_DIM, HID), lambda i: (0, 0)),
            pl.BlockSpec((1, HID), lambda i: (0, 0)),
        ],
        out_specs=pl.BlockSpec((BE, HID), lambda i: (i, 0)),
    )(ea, w, b)


def _make_layer_body(first, last):
    def body(*refs):
        if last:
            (p_ref, z_ref, h_ref, w1_ref, b1_ref, lng_ref, lnb_ref,
             w2_ref, b2_ref, g2_ref, bb2_ref, lw_ref, lb_ref, oh_ref) = refs
        else:
            (p_ref, z_ref, h_ref, w1_ref, b1_ref, lng_ref, lnb_ref,
             w2_ref, b2_ref, g2_ref, bb2_ref, oh_ref, oz_ref) = refs
        p = p_ref[...]
        acc = p[0] + p[1]
        den = acc[:, :HID]
        num = acc[:, HID:]
        z = z_ref[...]
        out = num / (den + 1e-16) + z
        u = jnp.dot(out, w1_ref[...], preferred_element_type=jnp.float32)
        u = jnp.maximum(_ln(u + b1_ref[...], lng_ref[...], lnb_ref[...]), 0.0)
        v = jnp.dot(u, w2_ref[...], preferred_element_type=jnp.float32)
        v = v + b2_ref[...]
        h_new = v if first else h_ref[...] + v
        zn = jnp.maximum(_ln(h_new, g2_ref[...], bb2_ref[...]), 0.0)
        if last:
            oh_ref[...] = jnp.dot(zn, lw_ref[...],
                                  preferred_element_type=jnp.float32) + lb_ref[...]
        else:
            oh_ref[...] = h_new
            oz_ref[...] = zn
    return body


def _layer_tc(partials, z, h, w1, b1, lng, lnb, w2, b2, g2, bb2,
              first=False, last=False, lw=None, lb=None):
    p3 = partials.reshape(NC, NP, 128)
    node_spec = pl.BlockSpec((BN, HID), lambda i: (i, 0))
    small = lambda r, c: pl.BlockSpec((r, c), lambda i: (0, 0))
    in_specs = [
        pl.BlockSpec((NC, BN, 128), lambda i: (0, i, 0)),
        node_spec,
        node_spec,
        small(HID, EXP),
        small(1, EXP),
        small(1, EXP),
        small(1, EXP),
        small(EXP, HID),
        small(1, HID),
        small(1, HID),
        small(1, HID),
    ]
    args = [p3, z, h, w1, b1, lng, lnb, w2, b2, g2, bb2]
    if last:
        in_specs += [small(HID, HID), small(1, HID)]
        args += [lw, lb]
        out_shape = jax.ShapeDtypeStruct((N, HID), jnp.float32)
        out_specs = node_spec
    else:
        out_shape = (jax.ShapeDtypeStruct((N, HID), jnp.float32),
                     jax.ShapeDtypeStruct((N, HID), jnp.float32))
        out_specs = (node_spec, node_spec)
    return pl.pallas_call(
        _make_layer_body(first, last),
        out_shape=out_shape,
        grid=(NB,),
        in_specs=in_specs,
        out_specs=out_specs,
    )(*args)


def kernel(x, edge_attr, node_W, node_b, edge_W, edge_b, t, conv_W1, conv_b1,
           conv_ln_g, conv_ln_b, conv_W2, conv_b2, layer_ln_g, layer_ln_b,
           lin_W, lin_b, edge_index):
    src = edge_index[0]
    dst = edge_index[1]
    zero_block = jnp.zeros((RPW, 128), jnp.float32)

    h0 = _node_encode(x, node_W, node_b.reshape(1, HID))
    ea = _edge_encode(edge_attr, edge_W, edge_b.reshape(1, HID))

    def r2(a):
        return a.reshape(1, -1)

    h = None
    z = h0
    for i in range(L):
        tvec = jnp.broadcast_to(t[i], (16,)).astype(jnp.float32)
        partials = _sc_aggregate(z, ea, src, dst, tvec, zero_block)
        first = (i == 0)
        last = (i == L - 1)
        g2 = layer_ln_g[0] if last else layer_ln_g[i + 1]
        bb2 = layer_ln_b[0] if last else layer_ln_b[i + 1]
        res = _layer_tc(
            partials, z, (z if first else h),
            conv_W1[i], r2(conv_b1[i]), r2(conv_ln_g[i]), r2(conv_ln_b[i]),
            conv_W2[i], r2(conv_b2[i]), r2(g2), r2(bb2),
            first=first, last=last,
            lw=(lin_W if last else None),
            lb=(r2(lin_b) if last else None),
        )
        if last:
            return res
        h, z = res


# speedup vs baseline: 20.9739x; 5.1290x over previous
import functools

import jax
import jax.numpy as jnp
from jax import lax
from jax.experimental import pallas as pl
from jax.experimental.pallas import tpu as pltpu
from jax.experimental.pallas import tpu_sc as plsc

N = 10000
E = 320000
NUM_FEAT = 128
HID = 64
EXP = 128
L = 12
EPS = 1e-7

NC = 2
NS = 16
NW = NC * NS
EPW = E // NW
CH = 80
NCHUNK = EPW // CH
NP = 10240
RPW = NP // NS


def _sc_agg_body(ztab_ref, ea_ref, src_ref, dst_ref, t_ref, zero_ref,
                 out_ref, src_all, d0, d1, h0, h1, e0, e1, ob, tv, acc,
                 sem0, sem1):
    cid = lax.axis_index("c")
    sid = lax.axis_index("s")
    wid = cid * NS + sid

    pltpu.sync_copy(t_ref, tv)
    pltpu.sync_copy(src_ref.at[pl.ds(wid * EPW, EPW)], src_all)
    pltpu.sync_copy(zero_ref, acc.at[pl.ds(sid * RPW, RPW)])
    plsc.subcore_barrier()

    tval = tv[...]
    hbufs = (h0, h1)
    ebufs = (e0, e1)
    dbufs = (d0, d1)
    sems = (sem0, sem1)

    def issue(c, b):
        pltpu.async_copy(ztab_ref.at[src_all.at[pl.ds(c * CH, CH)]],
                         hbufs[b], sems[b])
        pltpu.async_copy(ea_ref.at[pl.ds(wid * EPW + c * CH, CH)],
                         ebufs[b], sems[b])
        pltpu.async_copy(dst_ref.at[pl.ds(wid * EPW + c * CH, CH)],
                         dbufs[b], sems[b])

    def process(c, b):
        pltpu.make_async_copy(ztab_ref.at[pl.ds(0, CH)], hbufs[b], sems[b]).wait()
        pltpu.make_async_copy(ea_ref.at[pl.ds(0, CH)], ebufs[b], sems[b]).wait()
        pltpu.make_async_copy(dst_ref.at[pl.ds(0, CH)], dbufs[b], sems[b]).wait()
        hb = hbufs[b]
        eb = ebufs[b]

        @plsc.parallel_loop(0, CH, unroll=4)
        def _(r):
            for g in range(4):
                col = g * 16
                hv = hb[r, pl.ds(col, 16)]
                ev = eb[r, pl.ds(col, 16)]
                m = jnp.maximum(hv + ev, 0.0) + EPS
                ex = jnp.exp(tval * m)
                ob[r, pl.ds(col, 16)] = ex
                ob[r, pl.ds(64 + col, 16)] = ex * m

        pltpu.sync_copy(ob, acc.at[dbufs[b]], add=True)

    issue(0, 0)

    def it_body(it, _):
        c0 = it * 2
        issue(c0 + 1, 1)
        process(c0, 0)
        issue(c0 + 2, 0)
        process(c0 + 1, 1)
        return 0

    lax.fori_loop(0, (NCHUNK - 1) // 2, it_body, 0)
    process(NCHUNK - 1, 0)

    plsc.subcore_barrier()
    pltpu.sync_copy(acc.at[pl.ds(sid * RPW, RPW)],
                    out_ref.at[pl.ds(cid * NP + sid * RPW, RPW)])


def _sc_aggregate(ztab, ea, src, dst, tvec, zero_block):
    kern = pl.kernel(
        _sc_agg_body,
        out_type=jax.ShapeDtypeStruct((NC * NP, 128), jnp.float32),
        mesh=plsc.VectorSubcoreMesh(core_axis_name="c", subcore_axis_name="s",
                                    num_cores=NC, num_subcores=NS),
        scratch_types=[
            pltpu.VMEM((EPW,), jnp.int32),
            pltpu.VMEM((CH,), jnp.int32),
            pltpu.VMEM((CH,), jnp.int32),
            pltpu.VMEM((CH, HID), jnp.float32),
            pltpu.VMEM((CH, HID), jnp.float32),
            pltpu.VMEM((CH, HID), jnp.float32),
            pltpu.VMEM((CH, HID), jnp.float32),
            pltpu.VMEM((CH, 128), jnp.float32),
            pltpu.VMEM((16,), jnp.float32),
            pltpu.VMEM_SHARED((NP, 128), jnp.float32),
            pltpu.SemaphoreType.DMA,
            pltpu.SemaphoreType.DMA,
        ],
        compiler_params=pltpu.CompilerParams(use_tc_tiling_on_sc=False),
    )
    return kern(ztab, ea, src, dst, tvec, zero_block)


BN = 1000
NB = N // BN
EDGE_DIM = 4


def _ln(u, g, b):
    mu = jnp.mean(u, axis=-1, keepdims=True)
    var = jnp.mean((u - mu) ** 2, axis=-1, keepdims=True)
    return (u - mu) / jnp.sqrt(var + 1e-5) * g + b


def _node_encode_body(x_ref, w_ref, b_ref, o_ref):
    o_ref[...] = jnp.dot(x_ref[...], w_ref[...],
                         preferred_element_type=jnp.float32) + b_ref[...]


def _node_encode(x, w, b):
    return pl.pallas_call(
        _node_encode_body,
        out_shape=jax.ShapeDtypeStruct((N, HID), jnp.float32),
        grid=(NB,),
        in_specs=[
            pl.BlockSpec((BN, NUM_FEAT), lambda i: (i, 0)),
            pl.BlockSpec((NUM_FEAT, HID), lambda i: (0, 0)),
            pl.BlockSpec((1, HID), lambda i: (0, 0)),
        ],
        out_specs=pl.BlockSpec((BN, HID), lambda i: (i, 0)),
    )(x, w, b)


BE = 16000
NEB = E // BE


def _edge_encode_body(a_ref, w_ref, b_ref, o_ref):
    a = a_ref[...]
    w = w_ref[...]
    acc = jnp.broadcast_to(b_ref[...], (BE, HID))
    for k in range(EDGE_DIM):
        acc = acc + a[:, k:k + 1] * w[k:k + 1, :]
    o_ref[...] = acc


def _edge_encode(ea, w, b):
    return pl.pallas_call(
        _edge_encode_body,
        out_shape=jax.ShapeDtypeStruct((E, HID), jnp.float32),
        grid=(NEB,),
        in_specs=[
            pl.BlockSpec((BE, EDGE_DIM), lambda i: (i, 0)),
            pl.BlockSpec((EDGE_DIM, HID), lambda i: (0, 0)),
            pl.BlockSpec((1, HID), lambda i: (0, 0)),
        ],
        out_specs=pl.BlockSpec((BE, HID), lambda i: (i, 0)),
    )(ea, w, b)


def _make_layer_body(first, last):
    def body(*refs):
        if last:
            (p_ref, z_ref, h_ref, w1_ref, b1_ref, lng_ref, lnb_ref,
             w2_ref, b2_ref, g2_ref, bb2_ref, lw_ref, lb_ref, oh_ref) = refs
        else:
            (p_ref, z_ref, h_ref, w1_ref, b1_ref, lng_ref, lnb_ref,
             w2_ref, b2_ref, g2_ref, bb2_ref, oh_ref, oz_ref) = refs
        p = p_ref[...]
        acc = p[0] + p[1]
        den = acc[:, :HID]
        num = acc[:, HID:]
        z = z_ref[...]
        out = num / (den + 1e-16) + z
        u = jnp.dot(out, w1_ref[...], preferred_element_type=jnp.float32)
        u = jnp.maximum(_ln(u + b1_ref[...], lng_ref[...], lnb_ref[...]), 0.0)
        v = jnp.dot(u, w2_ref[...], preferred_element_type=jnp.float32)
        v = v + b2_ref[...]
        h_new = v if first else h_ref[...] + v
        zn = jnp.maximum(_ln(h_new, g2_ref[...], bb2_ref[...]), 0.0)
        if last:
            oh_ref[...] = jnp.dot(zn, lw_ref[...],
                                  preferred_element_type=jnp.float32) + lb_ref[...]
        else:
            oh_ref[...] = h_new
            oz_ref[...] = zn
    return body


def _layer_tc(partials, z, h, w1, b1, lng, lnb, w2, b2, g2, bb2,
              first=False, last=False, lw=None, lb=None):
    p3 = partials.reshape(NC, NP, 128)
    node_spec = pl.BlockSpec((BN, HID), lambda i: (i, 0))
    small = lambda r, c: pl.BlockSpec((r, c), lambda i: (0, 0))
    in_specs = [
        pl.BlockSpec((NC, BN, 128), lambda i: (0, i, 0)),
        node_spec,
        node_spec,
        small(HID, EXP),
        small(1, EXP),
        small(1, EXP),
        small(1, EXP),
        small(EXP, HID),
        small(1, HID),
        small(1, HID),
        small(1, HID),
    ]
    args = [p3, z, h, w1, b1, lng, lnb, w2, b2, g2, bb2]
    if last:
        in_specs += [small(HID, HID), small(1, HID)]
        args += [lw, lb]
        out_shape = jax.ShapeDtypeStruct((N, HID), jnp.float32)
        out_specs = node_spec
    else:
        out_shape = (jax.ShapeDtypeStruct((N, HID), jnp.float32),
                     jax.ShapeDtypeStruct((N, HID), jnp.float32))
        out_specs = (node_spec, node_spec)
    return pl.pallas_call(
        _make_layer_body(first, last),
        out_shape=out_shape,
        grid=(NB,),
        in_specs=in_specs,
        out_specs=out_specs,
    )(*args)


def kernel(x, edge_attr, node_W, node_b, edge_W, edge_b, t, conv_W1, conv_b1,
           conv_ln_g, conv_ln_b, conv_W2, conv_b2, layer_ln_g, layer_ln_b,
           lin_W, lin_b, edge_index):
    src = edge_index[0]
    dst = edge_index[1]
    zero_block = jnp.zeros((RPW, 128), jnp.float32)

    h0 = _node_encode(x, node_W, node_b.reshape(1, HID))
    ea = _edge_encode(edge_attr, edge_W, edge_b.reshape(1, HID))

    def r2(a):
        return a.reshape(1, -1)

    h = None
    z = h0
    for i in range(L):
        tvec = jnp.broadcast_to(t[i], (16,)).astype(jnp.float32)
        partials = _sc_aggregate(z, ea, src, dst, tvec, zero_block)
        first = (i == 0)
        last = (i == L - 1)
        g2 = layer_ln_g[0] if last else layer_ln_g[i + 1]
        bb2 = layer_ln_b[0] if last else layer_ln_b[i + 1]
        res = _layer_tc(
            partials, z, (z if first else h),
            conv_W1[i], r2(conv_b1[i]), r2(conv_ln_g[i]), r2(conv_ln_b[i]),
            conv_W2[i], r2(conv_b2[i]), r2(g2), r2(bb2),
            first=first, last=last,
            lw=(lin_W if last else None),
            lb=(r2(lin_b) if last else None),
        )
        if last:
            return res
        h, z = res
